# baseline (speedup 1.0000x reference)
import jax
import jax.numpy as jnp
from jax import lax
from jax.experimental import pallas as pl
from jax.experimental.pallas import tpu as pltpu
from jax.experimental.pallas import tpu_sc as plsc

NUM_WORKERS = 32
GROUP = 512


def kernel(token_ids, embedding_layer):
    n_rows, n_cols = token_ids.shape
    dim = embedding_layer.shape[1]
    num_indices = n_rows * n_cols
    idx = token_ids.reshape(num_indices)

    per_worker = num_indices // NUM_WORKERS
    n_groups = per_worker // GROUP

    mesh = plsc.VectorSubcoreMesh(core_axis_name="core",
                                  subcore_axis_name="subcore")

    @pl.kernel(
        out_type=jax.ShapeDtypeStruct((num_indices, dim), jnp.float32),
        mesh=mesh,
        compiler_params=pltpu.CompilerParams(use_tc_tiling_on_sc=False),
        scratch_types=[
            pltpu.VMEM((per_worker,), jnp.int32),
            pltpu.VMEM((GROUP, dim), jnp.float32),
            pltpu.VMEM((GROUP, dim), jnp.float32),
            pltpu.SemaphoreType.DMA,
            pltpu.SemaphoreType.DMA,
            pltpu.SemaphoreType.DMA,
            pltpu.SemaphoreType.DMA,
        ],
    )
    def gather_kernel(table_hbm, i_hbm, o_hbm, idx_all, rows_a, rows_b,
                      gsem_a, gsem_b, wsem_a, wsem_b):
        wid = lax.axis_index("subcore") * 2 + lax.axis_index("core")
        base = wid * per_worker
        pltpu.sync_copy(i_hbm.at[pl.ds(base, per_worker)], idx_all)

        def fire_gather(grp, rows, sem):
            return pltpu.async_copy(
                table_hbm.at[idx_all.at[pl.ds(grp * GROUP, GROUP)]],
                rows, sem)

        def fire_write(grp, rows, sem):
            return pltpu.async_copy(
                rows, o_hbm.at[pl.ds(base + grp * GROUP, GROUP)], sem)

        def drain(rows, sem):
            pltpu.make_async_copy(o_hbm.at[pl.ds(base, GROUP)], rows,
                                  sem).wait()

        fire_gather(0, rows_a, gsem_a)

        @pl.loop(0, n_groups, step=2)
        def _(gp):
            drain(rows_a, gsem_a)
            fire_write(gp, rows_a, wsem_a)
            db = fire_gather(gp + 1, rows_b, gsem_b)
            db.wait()
            fire_write(gp + 1, rows_b, wsem_b)
            drain(rows_a, wsem_a)

            @pl.when(gp + 2 < n_groups)
            def _():
                fire_gather(gp + 2, rows_a, gsem_a)

            drain(rows_b, wsem_b)

    out = gather_kernel(embedding_layer, idx)
    return out.reshape(n_rows, n_cols, dim)


# speedup vs baseline: 1.0011x; 1.0011x over previous
import jax
import jax.numpy as jnp
from jax import lax
from jax.experimental import pallas as pl
from jax.experimental.pallas import tpu as pltpu
from jax.experimental.pallas import tpu_sc as plsc

NUM_WORKERS = 32
WINDOW = 128
G = 8


def kernel(token_ids, embedding_layer):
    n_rows, n_cols = token_ids.shape
    dim = embedding_layer.shape[1]
    num_indices = n_rows * n_cols
    idx = token_ids.reshape(num_indices)

    per_worker = num_indices // NUM_WORKERS
    n_chunks = per_worker // WINDOW
    n_groups = n_chunks // G

    mesh = plsc.VectorSubcoreMesh(core_axis_name="core",
                                  subcore_axis_name="subcore")

    @pl.kernel(
        out_type=jax.ShapeDtypeStruct((num_indices, dim), jnp.float32),
        mesh=mesh,
        compiler_params=pltpu.CompilerParams(use_tc_tiling_on_sc=False),
        scratch_types=[
            pltpu.VMEM((per_worker,), jnp.int32),
            pltpu.VMEM((G, WINDOW, dim), jnp.float32),
            pltpu.SemaphoreType.DMA,
            pltpu.SemaphoreType.DMA,
        ],
    )
    def gather_kernel(table_hbm, i_hbm, o_hbm, idx_all, rows_v, gsem, wsem):
        wid = lax.axis_index("subcore") * 2 + lax.axis_index("core")
        base = wid * per_worker
        pltpu.sync_copy(i_hbm.at[pl.ds(base, per_worker)], idx_all)

        @pl.loop(0, n_groups)
        def _(grp):
            goff = grp * (G * WINDOW)
            gathers = []
            for j in range(G):
                gathers.append(pltpu.async_copy(
                    table_hbm.at[idx_all.at[pl.ds(goff + j * WINDOW, WINDOW)]],
                    rows_v.at[j], gsem))
            writes = []
            for j in range(G):
                gathers[j].wait()
                writes.append(pltpu.async_copy(
                    rows_v.at[j],
                    o_hbm.at[pl.ds(base + goff + j * WINDOW, WINDOW)], wsem))
            for w in writes:
                w.wait()

    out = gather_kernel(embedding_layer, idx)
    return out.reshape(n_rows, n_cols, dim)
